# Initial kernel scaffold; baseline (speedup 1.0000x reference)
import functools

import jax
import jax.numpy as jnp
from jax import lax
from jax.experimental import pallas as pl
from jax.experimental.pallas import tpu as pltpu
from jax.experimental.pallas import tpu_sc as plsc

_N = 10000
_E = 160000
_D = 256
_H = 256

_NS = 16
_K = 80
_ET = _E // _NS
_NCH = _ET // _K
_RT = _N // _NS

_BN = 1000
_GRID = _N // _BN


def _make_agg(with_count):
    mesh = plsc.VectorSubcoreMesh(core_axis_name="c", subcore_axis_name="s")
    out_type = [
        jax.ShapeDtypeStruct((_N, 128), jnp.float32),
        jax.ShapeDtypeStruct((_N, 128), jnp.float32),
    ]
    if with_count:
        out_type.append(jax.ShapeDtypeStruct((_N, 16), jnp.float32))

    scratch = [
        pltpu.VMEM((_K,), jnp.int32),
        pltpu.VMEM((_K,), jnp.int32),
        pltpu.VMEM((_K, 128), jnp.float32),
        pltpu.VMEM((_K, 16), jnp.float32),
        pltpu.VMEM_SHARED((_N, 128), jnp.float32),
        pltpu.VMEM_SHARED((_N, 16), jnp.float32),
        pltpu.SemaphoreType.DMA,
    ]

    def body(xs_h, src_h, dst_h, zrow_h, zcnt_h, ones_h, *rest):
        if with_count:
            outL, outR, outC, srcv, dstv, rows, ones, acc, cnt, sem = rest
        else:
            outL, outR, srcv, dstv, rows, ones, acc, cnt, sem = rest
        c = lax.axis_index("c")
        s = lax.axis_index("s")

        pltpu.sync_copy(zrow_h, acc.at[pl.ds(s * _RT, _RT)])
        if with_count:
            @pl.when(c == 0)
            def _():
                pltpu.sync_copy(zcnt_h, cnt.at[pl.ds(s * _RT, _RT)])
                pltpu.sync_copy(ones_h, ones)
        plsc.subcore_barrier()

        base0 = s * _ET
        off = c * _N

        def chunk(i, carry):
            base = pl.multiple_of(base0 + i * _K, 8)
            pltpu.sync_copy(src_h.at[pl.ds(base, _K)], srcv)
            pltpu.sync_copy(dst_h.at[pl.ds(base, _K)], dstv)
            for j in range(_K // 16):
                sl = pl.ds(j * 16, 16)
                srcv[sl] = srcv[sl] + off
            pltpu.async_copy(xs_h.at[srcv], rows, sem).wait()
            pltpu.sync_copy(rows, acc.at[dstv], add=True)
            if with_count:
                @pl.when(c == 0)
                def _():
                    pltpu.sync_copy(ones, cnt.at[dstv], add=True)
            return carry

        lax.fori_loop(0, _NCH, chunk, 0)
        plsc.subcore_barrier()

        rs = pl.ds(s * _RT, _RT)

        @pl.when(c == 0)
        def _():
            pltpu.sync_copy(acc.at[rs], outL.at[rs])
            if with_count:
                pltpu.sync_copy(cnt.at[rs], outC.at[rs])

        @pl.when(c == 1)
        def _():
            pltpu.sync_copy(acc.at[rs], outR.at[rs])

    return pl.kernel(body, out_type=out_type, mesh=mesh,
                     scratch_types=scratch)


_agg_count = _make_agg(True)
_agg_plain = _make_agg(False)


def _dot_t(a, w):
    return lax.dot_general(a, w, (((1,), (1,)), ((), ())),
                           preferred_element_type=jnp.float32)


def _dense1_body(aggL_ref, aggR_ref, cnt_ref, x_ref, wl_ref, bl_ref, wr_ref,
                 hs_ref, h_ref):
    inv = 1.0 / jnp.maximum(cnt_ref[:, 0:1], 1.0)
    aggL = aggL_ref[:, :] * inv
    aggR = aggR_ref[:, :] * inv
    agg = jnp.concatenate([aggL, aggR], axis=1)
    h = _dot_t(agg, wl_ref[:, :]) + bl_ref[:, :] + _dot_t(x_ref[:, :], wr_ref[:, :])
    h = jnp.maximum(h, 0.0)
    h_ref[:, :] = h
    hs_ref[0, :, :] = h[:, :128]
    hs_ref[1, :, :] = h[:, 128:]


_dense1 = pl.pallas_call(
    _dense1_body,
    grid=(_GRID,),
    in_specs=[
        pl.BlockSpec((_BN, 128), lambda i: (i, 0)),
        pl.BlockSpec((_BN, 128), lambda i: (i, 0)),
        pl.BlockSpec((_BN, 16), lambda i: (i, 0)),
        pl.BlockSpec((_BN, _D), lambda i: (i, 0)),
        pl.BlockSpec((_H, _D), lambda i: (0, 0)),
        pl.BlockSpec((1, _H), lambda i: (0, 0)),
        pl.BlockSpec((_H, _D), lambda i: (0, 0)),
    ],
    out_specs=[
        pl.BlockSpec((2, _BN, 128), lambda i: (0, i, 0)),
        pl.BlockSpec((_BN, _H), lambda i: (i, 0)),
    ],
    out_shape=[
        jax.ShapeDtypeStruct((2, _N, 128), jnp.float32),
        jax.ShapeDtypeStruct((_N, _H), jnp.float32),
    ],
)


def _dense2_body(aggL_ref, aggR_ref, cnt_ref, h1_ref, w2l_ref, b2l_ref,
                 w2r_ref, wl1_ref, bl1_ref, wl2_ref, bl2_ref, out_ref):
    inv = 1.0 / jnp.maximum(cnt_ref[:, 0:1], 1.0)
    agg = jnp.concatenate([aggL_ref[:, :] * inv, aggR_ref[:, :] * inv], axis=1)
    h = _dot_t(agg, w2l_ref[:, :]) + b2l_ref[:, :] + _dot_t(h1_ref[:, :], w2r_ref[:, :])
    h = jnp.maximum(h, 0.0)
    h = _dot_t(h, wl1_ref[:, :]) + bl1_ref[:, :]
    h = jnp.maximum(h, 0.0)
    out_ref[:, :] = _dot_t(h, wl2_ref[:, :]) + bl2_ref[:, :]


_dense2 = pl.pallas_call(
    _dense2_body,
    grid=(_GRID,),
    in_specs=[
        pl.BlockSpec((_BN, 128), lambda i: (i, 0)),
        pl.BlockSpec((_BN, 128), lambda i: (i, 0)),
        pl.BlockSpec((_BN, 16), lambda i: (i, 0)),
        pl.BlockSpec((_BN, _H), lambda i: (i, 0)),
        pl.BlockSpec((_H, _H), lambda i: (0, 0)),
        pl.BlockSpec((1, _H), lambda i: (0, 0)),
        pl.BlockSpec((_H, _H), lambda i: (0, 0)),
        pl.BlockSpec((_H, _H), lambda i: (0, 0)),
        pl.BlockSpec((1, _H), lambda i: (0, 0)),
        pl.BlockSpec((1, _H), lambda i: (0, 0)),
        pl.BlockSpec((1, 1), lambda i: (0, 0)),
    ],
    out_specs=pl.BlockSpec((_BN, 1), lambda i: (i, 0)),
    out_shape=jax.ShapeDtypeStruct((_N, 1), jnp.float32),
)


def kernel(x, edge_index, W1l, b1l, W1r, W2l, b2l, W2r, Wlin1, blin1, Wlin2,
           blin2):
    src = edge_index[0]
    dst = edge_index[1]
    xs = x.reshape(_N, 2, 128).transpose(1, 0, 2).reshape(2 * _N, 128)
    zrow = jnp.zeros((_RT, 128), jnp.float32)
    zcnt = jnp.zeros((_RT, 16), jnp.float32)
    ones = jnp.ones((_K, 16), jnp.float32)

    aggL, aggR, cntc = _agg_count(xs, src, dst, zrow, zcnt, ones)
    h1s, h1 = _dense1(aggL, aggR, cntc, x, W1l, b1l.reshape(1, _H), W1r)
    a2L, a2R = _agg_plain(h1s.reshape(2 * _N, 128), src, dst, zrow, zcnt, ones)
    lg = _dense2(a2L, a2R, cntc, h1, W2l, b2l.reshape(1, _H), W2r,
                 Wlin1, blin1.reshape(1, _H), Wlin2, blin2.reshape(1, 1))
    return lg[:, 0]


# speedup vs baseline: 1.8708x; 1.8708x over previous
import jax
import jax.numpy as jnp
from jax import lax
from jax.experimental import pallas as pl
from jax.experimental.pallas import tpu as pltpu
from jax.experimental.pallas import tpu_sc as plsc

_N = 10000
_E = 160000
_D = 256
_H = 256

_NS = 16
_K = 80
_ET = _E // _NS
_NCH = _ET // _K
_NP = 10240
_RT = _NP // _NS

_BN = 1000
_GRID = _N // _BN

_mesh = plsc.VectorSubcoreMesh(core_axis_name="c", subcore_axis_name="s",
                               num_cores=1)


def _agg_body(x_h, src_h, dst_h, zrow_h, out_h, srcv, dstv, rows, acc, sem):
    s = lax.axis_index("s")
    rs = pl.ds(s * _RT, _RT)
    pltpu.sync_copy(zrow_h, acc.at[rs])
    plsc.subcore_barrier()
    base0 = s * _ET

    def chunk(i, carry):
        base = pl.multiple_of(base0 + i * _K, 8)
        pltpu.sync_copy(src_h.at[pl.ds(base, _K)], srcv)
        pltpu.sync_copy(dst_h.at[pl.ds(base, _K)], dstv)
        pltpu.async_copy(x_h.at[srcv], rows, sem).wait()
        pltpu.sync_copy(rows, acc.at[dstv], add=True)
        return carry

    lax.fori_loop(0, _NCH, chunk, 0)
    plsc.subcore_barrier()
    pltpu.sync_copy(acc.at[rs], out_h.at[rs])


_agg = pl.kernel(
    _agg_body,
    out_type=[jax.ShapeDtypeStruct((_NP, 128), jnp.float32)],
    mesh=_mesh,
    scratch_types=[
        pltpu.VMEM((_K,), jnp.int32),
        pltpu.VMEM((_K,), jnp.int32),
        pltpu.VMEM((_K, 128), jnp.float32),
        pltpu.VMEM_SHARED((_NP, 128), jnp.float32),
        pltpu.SemaphoreType.DMA,
    ],
)


def _count_body(dst_h, zrow_h, ones_h, out_h, dstv, ones, cnt, sem):
    s = lax.axis_index("s")
    rs = pl.ds(s * _RT, _RT)
    pltpu.sync_copy(zrow_h, cnt.at[rs])
    pltpu.sync_copy(ones_h, ones)
    plsc.subcore_barrier()
    base0 = s * _ET

    def chunk(i, carry):
        base = pl.multiple_of(base0 + i * _K, 8)
        pltpu.sync_copy(dst_h.at[pl.ds(base, _K)], dstv)
        pltpu.sync_copy(ones, cnt.at[dstv], add=True)
        return carry

    lax.fori_loop(0, _NCH, chunk, 0)
    plsc.subcore_barrier()
    pltpu.sync_copy(cnt.at[rs], out_h.at[rs])


_count = pl.kernel(
    _count_body,
    out_type=[jax.ShapeDtypeStruct((_NP, 128), jnp.float32)],
    mesh=_mesh,
    scratch_types=[
        pltpu.VMEM((_K,), jnp.int32),
        pltpu.VMEM((_K, 128), jnp.float32),
        pltpu.VMEM_SHARED((_NP, 128), jnp.float32),
        pltpu.SemaphoreType.DMA,
    ],
)


def _dot_t(a, w):
    return lax.dot_general(a, w, (((1,), (1,)), ((), ())),
                           preferred_element_type=jnp.float32)


def _dense1_body(aggL_ref, aggR_ref, cnt_ref, x_ref, wl_ref, bl_ref, wr_ref,
                 hL_ref, hR_ref, h_ref):
    inv = 1.0 / jnp.maximum(cnt_ref[:, 0:1], 1.0)
    agg = jnp.concatenate([aggL_ref[:, :] * inv, aggR_ref[:, :] * inv], axis=1)
    h = _dot_t(agg, wl_ref[:, :]) + bl_ref[:, :] + _dot_t(x_ref[:, :], wr_ref[:, :])
    h = jnp.maximum(h, 0.0)
    h_ref[:, :] = h
    hL_ref[:, :] = h[:, :128]
    hR_ref[:, :] = h[:, 128:]


_dense1 = pl.pallas_call(
    _dense1_body,
    grid=(_GRID,),
    in_specs=[
        pl.BlockSpec((_BN, 128), lambda i: (i, 0)),
        pl.BlockSpec((_BN, 128), lambda i: (i, 0)),
        pl.BlockSpec((_BN, 128), lambda i: (i, 0)),
        pl.BlockSpec((_BN, _D), lambda i: (i, 0)),
        pl.BlockSpec((_H, _D), lambda i: (0, 0)),
        pl.BlockSpec((1, _H), lambda i: (0, 0)),
        pl.BlockSpec((_H, _D), lambda i: (0, 0)),
    ],
    out_specs=[
        pl.BlockSpec((_BN, 128), lambda i: (i, 0)),
        pl.BlockSpec((_BN, 128), lambda i: (i, 0)),
        pl.BlockSpec((_BN, _H), lambda i: (i, 0)),
    ],
    out_shape=[
        jax.ShapeDtypeStruct((_N, 128), jnp.float32),
        jax.ShapeDtypeStruct((_N, 128), jnp.float32),
        jax.ShapeDtypeStruct((_N, _H), jnp.float32),
    ],
)


def _dense2_body(aggL_ref, aggR_ref, cnt_ref, h1_ref, w2l_ref, b2l_ref,
                 w2r_ref, wl1_ref, bl1_ref, wl2_ref, bl2_ref, out_ref):
    inv = 1.0 / jnp.maximum(cnt_ref[:, 0:1], 1.0)
    agg = jnp.concatenate([aggL_ref[:, :] * inv, aggR_ref[:, :] * inv], axis=1)
    h = _dot_t(agg, w2l_ref[:, :]) + b2l_ref[:, :] + _dot_t(h1_ref[:, :], w2r_ref[:, :])
    h = jnp.maximum(h, 0.0)
    h = _dot_t(h, wl1_ref[:, :]) + bl1_ref[:, :]
    h = jnp.maximum(h, 0.0)
    lg = jnp.sum(h * wl2_ref[0:1, :], axis=1, keepdims=True)
    out_ref[:, :] = lg + bl2_ref[0, 0]


_dense2 = pl.pallas_call(
    _dense2_body,
    grid=(_GRID,),
    in_specs=[
        pl.BlockSpec((_BN, 128), lambda i: (i, 0)),
        pl.BlockSpec((_BN, 128), lambda i: (i, 0)),
        pl.BlockSpec((_BN, 128), lambda i: (i, 0)),
        pl.BlockSpec((_BN, _H), lambda i: (i, 0)),
        pl.BlockSpec((_H, _H), lambda i: (0, 0)),
        pl.BlockSpec((1, _H), lambda i: (0, 0)),
        pl.BlockSpec((_H, _H), lambda i: (0, 0)),
        pl.BlockSpec((_H, _H), lambda i: (0, 0)),
        pl.BlockSpec((1, _H), lambda i: (0, 0)),
        pl.BlockSpec((1, _H), lambda i: (0, 0)),
        pl.BlockSpec((1, 1), lambda i: (0, 0)),
    ],
    out_specs=pl.BlockSpec((_BN, 1), lambda i: (i, 0)),
    out_shape=jax.ShapeDtypeStruct((_N, 1), jnp.float32),
)


def kernel(x, edge_index, W1l, b1l, W1r, W2l, b2l, W2r, Wlin1, blin1, Wlin2,
           blin2):
    src = edge_index[0]
    dst = edge_index[1]
    xL = x[:, :128]
    xR = x[:, 128:]
    zrow = jnp.zeros((_RT, 128), jnp.float32)
    ones = jnp.ones((_K, 128), jnp.float32)

    cntc, = _count(dst, zrow, ones)
    aL, = _agg(xL, src, dst, zrow)
    aR, = _agg(xR, src, dst, zrow)
    h1L, h1R, h1 = _dense1(aL, aR, cntc, x, W1l, b1l.reshape(1, _H), W1r)
    a2L, = _agg(h1L, src, dst, zrow)
    a2R, = _agg(h1R, src, dst, zrow)
    lg = _dense2(a2L, a2R, cntc, h1, W2l, b2l.reshape(1, _H), W2r,
                 Wlin1, blin1.reshape(1, _H), Wlin2, blin2.reshape(1, 1))
    return lg[:, 0]


# speedup vs baseline: 2.0453x; 1.0933x over previous
import jax
import jax.numpy as jnp
from jax import lax
from jax.experimental import pallas as pl
from jax.experimental.pallas import tpu as pltpu
from jax.experimental.pallas import tpu_sc as plsc

_N = 10000
_E = 160000
_D = 256
_H = 256

_NS = 16
_K = 40
_ET = _E // _NS
_NCH = _ET // _K
_NP = 10240
_RT = _NP // _NS

_BN = 1000
_GRID = _N // _BN

_mesh = plsc.VectorSubcoreMesh(core_axis_name="c", subcore_axis_name="s",
                               num_cores=1)


_NB = 5
_RINGIT = _NCH // _NB - 1


def _agg_body(x_h, src_h, dst_h, zrow_h, out_h, *rest):
    srcv = rest[0:_NB]
    dstv = rest[_NB:2 * _NB]
    rows = rest[2 * _NB:3 * _NB]
    acc = rest[3 * _NB]
    sems = rest[3 * _NB + 1:]
    s = lax.axis_index("s")
    rs = pl.ds(s * _RT, _RT)
    pltpu.sync_copy(zrow_h, acc.at[rs])
    plsc.subcore_barrier()
    base0 = s * _ET

    for b in range(_NB):
        base = pl.multiple_of(base0 + b * _K, 8)
        pltpu.sync_copy(src_h.at[pl.ds(base, _K)], srcv[b])
        pltpu.sync_copy(dst_h.at[pl.ds(base, _K)], dstv[b])
        pltpu.async_copy(x_h.at[srcv[b]], rows[b], sems[b])

    def ring(r, carry):
        for b in range(_NB):
            pltpu.make_async_copy(x_h.at[srcv[b]], rows[b], sems[b]).wait()
            pltpu.sync_copy(rows[b], acc.at[dstv[b]], add=True)
            base = pl.multiple_of(base0 + ((r + 1) * _NB + b) * _K, 8)
            pltpu.sync_copy(src_h.at[pl.ds(base, _K)], srcv[b])
            pltpu.sync_copy(dst_h.at[pl.ds(base, _K)], dstv[b])
            pltpu.async_copy(x_h.at[srcv[b]], rows[b], sems[b])
        return carry

    lax.fori_loop(0, _RINGIT, ring, 0)

    for b in range(_NB):
        pltpu.make_async_copy(x_h.at[srcv[b]], rows[b], sems[b]).wait()
        pltpu.sync_copy(rows[b], acc.at[dstv[b]], add=True)

    plsc.subcore_barrier()
    pltpu.sync_copy(acc.at[rs], out_h.at[rs])


_agg = pl.kernel(
    _agg_body,
    out_type=[jax.ShapeDtypeStruct((_NP, 128), jnp.float32)],
    mesh=_mesh,
    scratch_types=(
        [pltpu.VMEM((_K,), jnp.int32)] * _NB
        + [pltpu.VMEM((_K,), jnp.int32)] * _NB
        + [pltpu.VMEM((_K, 128), jnp.float32)] * _NB
        + [pltpu.VMEM_SHARED((_NP, 128), jnp.float32)]
        + [pltpu.SemaphoreType.DMA] * _NB
    ),
)


def _count_body(dst_h, zrow_h, ones_h, out_h, *rest):
    dstv = rest[0:_NB]
    ones = rest[_NB]
    cnt = rest[_NB + 1]
    sems = rest[_NB + 2:]
    s = lax.axis_index("s")
    rs = pl.ds(s * _RT, _RT)
    pltpu.sync_copy(zrow_h, cnt.at[rs])
    pltpu.sync_copy(ones_h, ones)
    plsc.subcore_barrier()
    base0 = s * _ET

    for b in range(_NB):
        base = pl.multiple_of(base0 + b * _K, 8)
        pltpu.async_copy(dst_h.at[pl.ds(base, _K)], dstv[b], sems[b])

    def ring(r, carry):
        for b in range(_NB):
            pltpu.make_async_copy(dst_h.at[pl.ds(base0, _K)], dstv[b],
                                  sems[b]).wait()
            pltpu.sync_copy(ones, cnt.at[dstv[b]], add=True)
            base = pl.multiple_of(base0 + ((r + 1) * _NB + b) * _K, 8)
            pltpu.async_copy(dst_h.at[pl.ds(base, _K)], dstv[b], sems[b])
        return carry

    lax.fori_loop(0, _RINGIT, ring, 0)

    for b in range(_NB):
        pltpu.make_async_copy(dst_h.at[pl.ds(base0, _K)], dstv[b],
                              sems[b]).wait()
        pltpu.sync_copy(ones, cnt.at[dstv[b]], add=True)

    plsc.subcore_barrier()
    pltpu.sync_copy(cnt.at[rs], out_h.at[rs])


_count = pl.kernel(
    _count_body,
    out_type=[jax.ShapeDtypeStruct((_NP, 128), jnp.float32)],
    mesh=_mesh,
    scratch_types=(
        [pltpu.VMEM((_K,), jnp.int32)] * _NB
        + [pltpu.VMEM((_K, 128), jnp.float32)]
        + [pltpu.VMEM_SHARED((_NP, 128), jnp.float32)]
        + [pltpu.SemaphoreType.DMA] * _NB
    ),
)


def _dot_t(a, w):
    return lax.dot_general(a, w, (((1,), (1,)), ((), ())),
                           preferred_element_type=jnp.float32)


def _dense1_body(aggL_ref, aggR_ref, cnt_ref, x_ref, wl_ref, bl_ref, wr_ref,
                 hL_ref, hR_ref, h_ref):
    inv = 1.0 / jnp.maximum(cnt_ref[:, 0:1], 1.0)
    agg = jnp.concatenate([aggL_ref[:, :] * inv, aggR_ref[:, :] * inv], axis=1)
    h = _dot_t(agg, wl_ref[:, :]) + bl_ref[:, :] + _dot_t(x_ref[:, :], wr_ref[:, :])
    h = jnp.maximum(h, 0.0)
    h_ref[:, :] = h
    hL_ref[:, :] = h[:, :128]
    hR_ref[:, :] = h[:, 128:]


_dense1 = pl.pallas_call(
    _dense1_body,
    grid=(_GRID,),
    in_specs=[
        pl.BlockSpec((_BN, 128), lambda i: (i, 0)),
        pl.BlockSpec((_BN, 128), lambda i: (i, 0)),
        pl.BlockSpec((_BN, 128), lambda i: (i, 0)),
        pl.BlockSpec((_BN, _D), lambda i: (i, 0)),
        pl.BlockSpec((_H, _D), lambda i: (0, 0)),
        pl.BlockSpec((1, _H), lambda i: (0, 0)),
        pl.BlockSpec((_H, _D), lambda i: (0, 0)),
    ],
    out_specs=[
        pl.BlockSpec((_BN, 128), lambda i: (i, 0)),
        pl.BlockSpec((_BN, 128), lambda i: (i, 0)),
        pl.BlockSpec((_BN, _H), lambda i: (i, 0)),
    ],
    out_shape=[
        jax.ShapeDtypeStruct((_N, 128), jnp.float32),
        jax.ShapeDtypeStruct((_N, 128), jnp.float32),
        jax.ShapeDtypeStruct((_N, _H), jnp.float32),
    ],
)


def _dense2_body(aggL_ref, aggR_ref, cnt_ref, h1_ref, w2l_ref, b2l_ref,
                 w2r_ref, wl1_ref, bl1_ref, wl2_ref, bl2_ref, out_ref):
    inv = 1.0 / jnp.maximum(cnt_ref[:, 0:1], 1.0)
    agg = jnp.concatenate([aggL_ref[:, :] * inv, aggR_ref[:, :] * inv], axis=1)
    h = _dot_t(agg, w2l_ref[:, :]) + b2l_ref[:, :] + _dot_t(h1_ref[:, :], w2r_ref[:, :])
    h = jnp.maximum(h, 0.0)
    h = _dot_t(h, wl1_ref[:, :]) + bl1_ref[:, :]
    h = jnp.maximum(h, 0.0)
    lg = jnp.sum(h * wl2_ref[0:1, :], axis=1, keepdims=True)
    out_ref[:, :] = lg + bl2_ref[0, 0]


_dense2 = pl.pallas_call(
    _dense2_body,
    grid=(_GRID,),
    in_specs=[
        pl.BlockSpec((_BN, 128), lambda i: (i, 0)),
        pl.BlockSpec((_BN, 128), lambda i: (i, 0)),
        pl.BlockSpec((_BN, 128), lambda i: (i, 0)),
        pl.BlockSpec((_BN, _H), lambda i: (i, 0)),
        pl.BlockSpec((_H, _H), lambda i: (0, 0)),
        pl.BlockSpec((1, _H), lambda i: (0, 0)),
        pl.BlockSpec((_H, _H), lambda i: (0, 0)),
        pl.BlockSpec((_H, _H), lambda i: (0, 0)),
        pl.BlockSpec((1, _H), lambda i: (0, 0)),
        pl.BlockSpec((1, _H), lambda i: (0, 0)),
        pl.BlockSpec((1, 1), lambda i: (0, 0)),
    ],
    out_specs=pl.BlockSpec((_BN, 1), lambda i: (i, 0)),
    out_shape=jax.ShapeDtypeStruct((_N, 1), jnp.float32),
)


def kernel(x, edge_index, W1l, b1l, W1r, W2l, b2l, W2r, Wlin1, blin1, Wlin2,
           blin2):
    src = edge_index[0]
    dst = edge_index[1]
    xL = x[:, :128]
    xR = x[:, 128:]
    zrow = jnp.zeros((_RT, 128), jnp.float32)
    ones = jnp.ones((_K, 128), jnp.float32)

    cntc, = _count(dst, zrow, ones)
    aL, = _agg(xL, src, dst, zrow)
    aR, = _agg(xR, src, dst, zrow)
    h1L, h1R, h1 = _dense1(aL, aR, cntc, x, W1l, b1l.reshape(1, _H), W1r)
    a2L, = _agg(h1L, src, dst, zrow)
    a2R, = _agg(h1R, src, dst, zrow)
    lg = _dense2(a2L, a2R, cntc, h1, W2l, b2l.reshape(1, _H), W2r,
                 Wlin1, blin1.reshape(1, _H), Wlin2, blin2.reshape(1, 1))
    return lg[:, 0]


# speedup vs baseline: 2.1402x; 1.0464x over previous
import jax
import jax.numpy as jnp
from jax import lax
from jax.experimental import pallas as pl
from jax.experimental.pallas import tpu as pltpu
from jax.experimental.pallas import tpu_sc as plsc

_N = 10000
_E = 160000
_D = 256
_H = 256

_NS = 16
_K = 128
_EP = 163840
_CR = _EP // _K
_CT = _CR // _NS
_SCH = 16
_SUP = _CT // _SCH
_NB = 2
_NP = 10112
_RT = _NP // _NS
_TRASH = 10100

_BN = 1000
_GRID = _N // _BN

_mesh = plsc.VectorSubcoreMesh(core_axis_name="c", subcore_axis_name="s",
                               num_cores=1)


def _agg_body(x_h, src_h, dst_h, zrow_h, out_h, *rest):
    sblk = rest[0:2]
    dblk = rest[2:4]
    rows = rest[4:4 + _NB]
    acc = rest[4 + _NB]
    semI = rest[5 + _NB:7 + _NB]
    semG = rest[7 + _NB:]
    s = lax.axis_index("s")
    rs = pl.ds(s * _RT, _RT)
    pltpu.sync_copy(zrow_h, acc.at[rs])
    row0 = s * _CT

    pltpu.sync_copy(src_h.at[pl.ds(row0, _SCH)], sblk[0])
    pltpu.sync_copy(dst_h.at[pl.ds(row0, _SCH)], dblk[0])
    plsc.subcore_barrier()

    for si in range(_SUP):
        p = si % 2
        q = 1 - p
        if si + 1 < _SUP:
            nxt = pl.ds(row0 + (si + 1) * _SCH, _SCH)
            pltpu.async_copy(src_h.at[nxt], sblk[q], semI[q])
            pltpu.async_copy(dst_h.at[nxt], dblk[q], semI[q])

        for b in range(_NB):
            pltpu.async_copy(x_h.at[sblk[p].at[b]], rows[b], semG[b])

        def mid(r, carry, p=p):
            for b in range(_NB):
                j = r * _NB + b
                pltpu.make_async_copy(x_h.at[sblk[p].at[j]], rows[b],
                                      semG[b]).wait()
                pltpu.sync_copy(rows[b], acc.at[dblk[p].at[j]], add=True)
                pltpu.async_copy(x_h.at[sblk[p].at[j + _NB]], rows[b],
                                 semG[b])
            return carry

        lax.fori_loop(0, _SCH // _NB - 1, mid, 0)

        for b in range(_NB):
            j = _SCH - _NB + b
            pltpu.make_async_copy(x_h.at[sblk[p].at[j]], rows[b],
                                  semG[b]).wait()
            pltpu.sync_copy(rows[b], acc.at[dblk[p].at[j]], add=True)

        if si + 1 < _SUP:
            pltpu.make_async_copy(src_h.at[nxt], sblk[q], semI[q]).wait()
            pltpu.make_async_copy(dst_h.at[nxt], dblk[q], semI[q]).wait()

    plsc.subcore_barrier()
    pltpu.sync_copy(acc.at[rs], out_h.at[rs])


_agg = pl.kernel(
    _agg_body,
    out_type=[jax.ShapeDtypeStruct((_NP, 128), jnp.float32)],
    mesh=_mesh,
    scratch_types=(
        [pltpu.VMEM((_SCH, _K), jnp.int32)] * 2
        + [pltpu.VMEM((_SCH, _K), jnp.int32)] * 2
        + [pltpu.VMEM((_K, 128), jnp.float32)] * _NB
        + [pltpu.VMEM_SHARED((_NP, 128), jnp.float32)]
        + [pltpu.SemaphoreType.DMA] * 2
        + [pltpu.SemaphoreType.DMA] * _NB
    ),
)


def _count_body(dst_h, zrow_h, ones_h, out_h, *rest):
    dblk = rest[0:2]
    ones = rest[2]
    cnt = rest[3]
    semI = rest[4:6]
    s = lax.axis_index("s")
    rs = pl.ds(s * _RT, _RT)
    pltpu.sync_copy(zrow_h, cnt.at[rs])
    pltpu.sync_copy(ones_h, ones)
    row0 = s * _CT
    pltpu.sync_copy(dst_h.at[pl.ds(row0, _SCH)], dblk[0])
    plsc.subcore_barrier()

    for si in range(_SUP):
        p = si % 2
        q = 1 - p
        if si + 1 < _SUP:
            nxt = pl.ds(row0 + (si + 1) * _SCH, _SCH)
            pltpu.async_copy(dst_h.at[nxt], dblk[q], semI[q])

        def mid(r, carry, p=p):
            pltpu.sync_copy(ones, cnt.at[dblk[p].at[r]], add=True)
            return carry

        lax.fori_loop(0, _SCH, mid, 0)

        if si + 1 < _SUP:
            pltpu.make_async_copy(dst_h.at[nxt], dblk[q], semI[q]).wait()

    plsc.subcore_barrier()
    pltpu.sync_copy(cnt.at[rs], out_h.at[rs])


_count = pl.kernel(
    _count_body,
    out_type=[jax.ShapeDtypeStruct((_NP, 128), jnp.float32)],
    mesh=_mesh,
    scratch_types=(
        [pltpu.VMEM((_SCH, _K), jnp.int32)] * 2
        + [pltpu.VMEM((_K, 128), jnp.float32)]
        + [pltpu.VMEM_SHARED((_NP, 128), jnp.float32)]
        + [pltpu.SemaphoreType.DMA] * 2
    ),
)


def _dot_t(a, w):
    return lax.dot_general(a, w, (((1,), (1,)), ((), ())),
                           preferred_element_type=jnp.float32)


def _dense1_body(aggL_ref, aggR_ref, cnt_ref, x_ref, wl_ref, bl_ref, wr_ref,
                 hL_ref, hR_ref, h_ref):
    inv = 1.0 / jnp.maximum(cnt_ref[:, 0:1], 1.0)
    agg = jnp.concatenate([aggL_ref[:, :] * inv, aggR_ref[:, :] * inv], axis=1)
    h = _dot_t(agg, wl_ref[:, :]) + bl_ref[:, :] + _dot_t(x_ref[:, :], wr_ref[:, :])
    h = jnp.maximum(h, 0.0)
    h_ref[:, :] = h
    hL_ref[:, :] = h[:, :128]
    hR_ref[:, :] = h[:, 128:]


_dense1 = pl.pallas_call(
    _dense1_body,
    grid=(_GRID,),
    in_specs=[
        pl.BlockSpec((_BN, 128), lambda i: (i, 0)),
        pl.BlockSpec((_BN, 128), lambda i: (i, 0)),
        pl.BlockSpec((_BN, 128), lambda i: (i, 0)),
        pl.BlockSpec((_BN, _D), lambda i: (i, 0)),
        pl.BlockSpec((_H, _D), lambda i: (0, 0)),
        pl.BlockSpec((1, _H), lambda i: (0, 0)),
        pl.BlockSpec((_H, _D), lambda i: (0, 0)),
    ],
    out_specs=[
        pl.BlockSpec((_BN, 128), lambda i: (i, 0)),
        pl.BlockSpec((_BN, 128), lambda i: (i, 0)),
        pl.BlockSpec((_BN, _H), lambda i: (i, 0)),
    ],
    out_shape=[
        jax.ShapeDtypeStruct((_N, 128), jnp.float32),
        jax.ShapeDtypeStruct((_N, 128), jnp.float32),
        jax.ShapeDtypeStruct((_N, _H), jnp.float32),
    ],
)


def _dense2_body(aggL_ref, aggR_ref, cnt_ref, h1_ref, w2l_ref, b2l_ref,
                 w2r_ref, wl1_ref, bl1_ref, wl2_ref, bl2_ref, out_ref):
    inv = 1.0 / jnp.maximum(cnt_ref[:, 0:1], 1.0)
    agg = jnp.concatenate([aggL_ref[:, :] * inv, aggR_ref[:, :] * inv], axis=1)
    h = _dot_t(agg, w2l_ref[:, :]) + b2l_ref[:, :] + _dot_t(h1_ref[:, :], w2r_ref[:, :])
    h = jnp.maximum(h, 0.0)
    h = _dot_t(h, wl1_ref[:, :]) + bl1_ref[:, :]
    h = jnp.maximum(h, 0.0)
    lg = jnp.sum(h * wl2_ref[0:1, :], axis=1, keepdims=True)
    out_ref[:, :] = lg + bl2_ref[0, 0]


_dense2 = pl.pallas_call(
    _dense2_body,
    grid=(_GRID,),
    in_specs=[
        pl.BlockSpec((_BN, 128), lambda i: (i, 0)),
        pl.BlockSpec((_BN, 128), lambda i: (i, 0)),
        pl.BlockSpec((_BN, 128), lambda i: (i, 0)),
        pl.BlockSpec((_BN, _H), lambda i: (i, 0)),
        pl.BlockSpec((_H, _H), lambda i: (0, 0)),
        pl.BlockSpec((1, _H), lambda i: (0, 0)),
        pl.BlockSpec((_H, _H), lambda i: (0, 0)),
        pl.BlockSpec((_H, _H), lambda i: (0, 0)),
        pl.BlockSpec((1, _H), lambda i: (0, 0)),
        pl.BlockSpec((1, _H), lambda i: (0, 0)),
        pl.BlockSpec((1, 1), lambda i: (0, 0)),
    ],
    out_specs=pl.BlockSpec((_BN, 1), lambda i: (i, 0)),
    out_shape=jax.ShapeDtypeStruct((_N, 1), jnp.float32),
)


def kernel(x, edge_index, W1l, b1l, W1r, W2l, b2l, W2r, Wlin1, blin1, Wlin2,
           blin2):
    npad = _EP - _E
    src2d = jnp.concatenate(
        [edge_index[0], jnp.zeros((npad,), jnp.int32)]).reshape(_CR, _K)
    dst2d = jnp.concatenate(
        [edge_index[1], jnp.full((npad,), _TRASH, jnp.int32)]).reshape(_CR, _K)
    xL = x[:, :128]
    xR = x[:, 128:]
    zrow = jnp.zeros((_RT, 128), jnp.float32)
    ones = jnp.ones((_K, 128), jnp.float32)

    cntc, = _count(dst2d, zrow, ones)
    aL, = _agg(xL, src2d, dst2d, zrow)
    aR, = _agg(xR, src2d, dst2d, zrow)
    h1L, h1R, h1 = _dense1(aL, aR, cntc, x, W1l, b1l.reshape(1, _H), W1r)
    a2L, = _agg(h1L, src2d, dst2d, zrow)
    a2R, = _agg(h1R, src2d, dst2d, zrow)
    lg = _dense2(a2L, a2R, cntc, h1, W2l, b2l.reshape(1, _H), W2r,
                 Wlin1, blin1.reshape(1, _H), Wlin2, blin2.reshape(1, 1))
    return lg[:, 0]
